# baseline (speedup 1.0000x reference)
import functools
import math

import jax
import jax.numpy as jnp
from jax import lax
from jax.experimental import pallas as pl
from jax.experimental.pallas import tpu as pltpu
from jax.experimental.pallas import tpu_sc as plsc

D_MODEL_DIM = 64
SCALE = math.sqrt(D_MODEL_DIM)

NC = 2
NS = 16
NW = NC * NS

IDX_PER_DMA = 128
CHUNK = 512
DMAS_PER_CHUNK = CHUNK // IDX_PER_DMA


def _make_lookup(B):
    b_per_w = B // NW
    idx_rows_per_w = b_per_w // IDX_PER_DMA
    n_chunks = b_per_w // CHUNK

    mesh = plsc.VectorSubcoreMesh(core_axis_name="c", subcore_axis_name="s")

    @functools.partial(
        pl.kernel,
        mesh=mesh,
        out_type=jax.ShapeDtypeStruct((B, D_MODEL_DIM), jnp.float32),
        compiler_params=pltpu.CompilerParams(use_tc_tiling_on_sc=False),
        scratch_types=[
            pltpu.VMEM((idx_rows_per_w, IDX_PER_DMA), jnp.int32),
            pltpu.VMEM((CHUNK, D_MODEL_DIM), jnp.float32),
            pltpu.SemaphoreType.DMA,
        ],
    )
    def lookup(x_hbm, table_hbm, out_hbm, idx_v, rows_v, gsem):
        wid = lax.axis_index("s") * NC + lax.axis_index("c")
        base = wid * b_per_w

        pltpu.sync_copy(x_hbm.at[pl.ds(wid * idx_rows_per_w, idx_rows_per_w)],
                        idx_v)

        def chunk_body(c, carry):
            descs = []
            for j in range(DMAS_PER_CHUNK):
                descs.append(pltpu.async_copy(
                    table_hbm.at[idx_v.at[c * DMAS_PER_CHUNK + j]],
                    rows_v.at[pl.ds(j * IDX_PER_DMA, IDX_PER_DMA)],
                    gsem,
                ))
            for d in descs:
                d.wait()

            def row_body(r, _):
                for dd in range(D_MODEL_DIM // 16):
                    sl = pl.ds(dd * 16, 16)
                    rows_v[r, sl] = rows_v[r, sl] * SCALE
                return _

            lax.fori_loop(0, CHUNK, row_body, 0, unroll=2)

            pltpu.sync_copy(rows_v, out_hbm.at[pl.ds(base + c * CHUNK, CHUNK)])
            return carry

        lax.fori_loop(0, n_chunks, chunk_body, 0)

    return lookup


def kernel(x, table):
    B = x.shape[0] * x.shape[1]
    xf = x.reshape(NW * (B // NW // IDX_PER_DMA), IDX_PER_DMA).astype(jnp.int32)
    out = _make_lookup(B)(xf, table)
    return out.reshape(x.shape + (table.shape[1],))


# speedup vs baseline: 1.0663x; 1.0663x over previous
import functools
import math

import jax
import jax.numpy as jnp
from jax import lax
from jax.experimental import pallas as pl
from jax.experimental.pallas import tpu as pltpu
from jax.experimental.pallas import tpu_sc as plsc

D_MODEL_DIM = 64
SCALE = math.sqrt(D_MODEL_DIM)

NC = 2
NS = 16
NW = NC * NS

IDX_PER_DMA = 128
CHUNK = 256
DMAS_PER_CHUNK = CHUNK // IDX_PER_DMA
NBUF = 4
LOOKAHEAD = 2


def _make_lookup(B):
    b_per_w = B // NW
    idx_rows_per_w = b_per_w // IDX_PER_DMA
    n_chunks = b_per_w // CHUNK
    n_super = n_chunks // NBUF

    mesh = plsc.VectorSubcoreMesh(core_axis_name="c", subcore_axis_name="s")

    @functools.partial(
        pl.kernel,
        mesh=mesh,
        out_type=jax.ShapeDtypeStruct((B, D_MODEL_DIM), jnp.float32),
        compiler_params=pltpu.CompilerParams(use_tc_tiling_on_sc=False),
        scratch_types=[
            pltpu.VMEM((idx_rows_per_w, IDX_PER_DMA), jnp.int32),
            pltpu.VMEM((NBUF, CHUNK, D_MODEL_DIM), jnp.float32),
        ]
        + [pltpu.SemaphoreType.DMA] * NBUF
        + [pltpu.SemaphoreType.DMA] * NBUF,
    )
    def lookup(x_hbm, table_hbm, out_hbm, idx_v, rows_v, *sems):
        gsems = sems[:NBUF]
        wsems = sems[NBUF:]
        wid = lax.axis_index("s") * NC + lax.axis_index("c")
        base = wid * b_per_w

        pltpu.sync_copy(x_hbm.at[pl.ds(wid * idx_rows_per_w, idx_rows_per_w)],
                        idx_v)

        def fire_gather(c, b):
            for j in range(DMAS_PER_CHUNK):
                pltpu.async_copy(
                    table_hbm.at[idx_v.at[c * DMAS_PER_CHUNK + j]],
                    rows_v.at[b, pl.ds(j * IDX_PER_DMA, IDX_PER_DMA)],
                    gsems[b],
                )

        def drain_gather(b):
            pltpu.make_async_copy(table_hbm.at[pl.ds(0, CHUNK)],
                                  rows_v.at[b], gsems[b]).wait()

        def drain_write(b):
            pltpu.make_async_copy(rows_v.at[b], out_hbm.at[pl.ds(0, CHUNK)],
                                  wsems[b]).wait()

        for c0 in range(LOOKAHEAD):
            fire_gather(c0, c0 % NBUF)

        @pl.loop(0, n_super)
        def super_step(s):
            c_base = s * NBUF
            for b0 in range(NBUF):
                c = c_base + b0
                b = b0
                bn = (b0 + LOOKAHEAD) % NBUF
                @pl.when(c + LOOKAHEAD < n_chunks)
                def _():
                    @pl.when(c >= NBUF - LOOKAHEAD)
                    def _():
                        drain_write(bn)
                    fire_gather(c + LOOKAHEAD, bn)

                drain_gather(b)

                @plsc.parallel_loop(0, CHUNK, unroll=4)
                def row_body(r):
                    for dd in range(D_MODEL_DIM // 16):
                        sl = pl.ds(dd * 16, 16)
                        rows_v[b, r, sl] = rows_v[b, r, sl] * SCALE

                pltpu.async_copy(rows_v.at[b],
                                 out_hbm.at[pl.ds(base + c * CHUNK, CHUNK)],
                                 wsems[b])

        for ct in range(n_chunks - NBUF, n_chunks):
            drain_write(ct % NBUF)

    return lookup


def kernel(x, table):
    B = x.shape[0] * x.shape[1]
    xf = x.reshape(NW * (B // NW // IDX_PER_DMA), IDX_PER_DMA).astype(jnp.int32)
    out = _make_lookup(B)(xf, table)
    return out.reshape(x.shape + (table.shape[1],))


# speedup vs baseline: 1.0667x; 1.0003x over previous
import functools
import math

import jax
import jax.numpy as jnp
from jax import lax
from jax.experimental import pallas as pl
from jax.experimental.pallas import tpu as pltpu
from jax.experimental.pallas import tpu_sc as plsc

D_MODEL_DIM = 64
SCALE = math.sqrt(D_MODEL_DIM)

NC = 2
NS = 16
NW = NC * NS

SEQ = 200
IDX_PER_DMA = 100
DMAS_PER_CHUNK = SEQ // IDX_PER_DMA
NBUF = 4
LOOKAHEAD = 2


def _make_lookup(n_rows):
    rows_per_w = n_rows // NW
    n_chunks = rows_per_w
    n_super = n_chunks // NBUF
    idx_rows_per_w = rows_per_w * DMAS_PER_CHUNK

    mesh = plsc.VectorSubcoreMesh(core_axis_name="c", subcore_axis_name="s")

    @functools.partial(
        pl.kernel,
        mesh=mesh,
        out_type=jax.ShapeDtypeStruct((n_rows, SEQ, D_MODEL_DIM), jnp.float32),
        compiler_params=pltpu.CompilerParams(use_tc_tiling_on_sc=False),
        scratch_types=[
            pltpu.VMEM((idx_rows_per_w, IDX_PER_DMA), jnp.int32),
            pltpu.VMEM((NBUF, 1, SEQ, D_MODEL_DIM), jnp.float32),
        ]
        + [pltpu.SemaphoreType.DMA] * NBUF
        + [pltpu.SemaphoreType.DMA] * NBUF,
    )
    def lookup(x_hbm, table_hbm, out_hbm, idx_v, rows_v, *sems):
        gsems = sems[:NBUF]
        wsems = sems[NBUF:]
        wid = lax.axis_index("s") * NC + lax.axis_index("c")
        xbase = wid * rows_per_w

        pltpu.sync_copy(x_hbm.at[pl.ds(wid * idx_rows_per_w, idx_rows_per_w)],
                        idx_v)

        def fire_gather(c, b):
            for j in range(DMAS_PER_CHUNK):
                pltpu.async_copy(
                    table_hbm.at[idx_v.at[c * DMAS_PER_CHUNK + j]],
                    rows_v.at[b, 0, pl.ds(j * IDX_PER_DMA, IDX_PER_DMA)],
                    gsems[b],
                )

        def drain_gather(b):
            pltpu.make_async_copy(table_hbm.at[pl.ds(0, SEQ)],
                                  rows_v.at[b, 0], gsems[b]).wait()

        def drain_write(b):
            pltpu.make_async_copy(rows_v.at[b], out_hbm.at[pl.ds(0, 1)],
                                  wsems[b]).wait()

        for c0 in range(LOOKAHEAD):
            fire_gather(c0, c0 % NBUF)

        @pl.loop(0, n_super)
        def super_step(s):
            c_base = s * NBUF
            for b0 in range(NBUF):
                c = c_base + b0
                b = b0
                bn = (b0 + LOOKAHEAD) % NBUF
                @pl.when(c + LOOKAHEAD < n_chunks)
                def _():
                    @pl.when(c >= NBUF - LOOKAHEAD)
                    def _():
                        drain_write(bn)
                    fire_gather(c + LOOKAHEAD, bn)

                drain_gather(b)

                @plsc.parallel_loop(0, SEQ, unroll=4)
                def row_body(r):
                    for dd in range(D_MODEL_DIM // 16):
                        sl = pl.ds(dd * 16, 16)
                        rows_v[b, 0, r, sl] = rows_v[b, 0, r, sl] * SCALE

                pltpu.async_copy(rows_v.at[b],
                                 out_hbm.at[pl.ds(xbase + c, 1)],
                                 wsems[b])

        for ct in range(n_chunks - NBUF, n_chunks):
            drain_write(ct % NBUF)

    return lookup


def kernel(x, table):
    n_rows = x.shape[0]
    xf = x.reshape(n_rows * DMAS_PER_CHUNK, IDX_PER_DMA).astype(jnp.int32)
    return _make_lookup(n_rows)(xf, table)
